# baseline (speedup 1.0000x reference)
import functools
import math

import jax
import jax.numpy as jnp
from jax import lax
from jax.experimental import pallas as pl
from jax.experimental.pallas import tpu as pltpu
from jax.experimental.pallas import tpu_sc as plsc

def _coef_body(em_ref, mo_ref, cf_ref):
    b, n = em_ref.shape
    no = n + 2
    em = em_ref[...]
    mo_ref[:, pl.ds(0, 2)] = jnp.ones((b, 2), jnp.int32)
    mo_ref[:, pl.ds(2, n)] = em

    emf = em.astype(jnp.float32)
    r_i = lax.broadcasted_iota(jnp.int32, (n, n), 0)
    c_i = lax.broadcasted_iota(jnp.int32, (n, n), 1)
    tri = (r_i >= c_i).astype(jnp.float32)
    s = jnp.dot(emf, tri, preferred_element_type=jnp.float32)
    h = emf * ((s - emf) == 0.0).astype(jnp.float32)
    allzero = (s[:, 0:1] == 0.0).astype(jnp.float32)

    zero1 = jnp.zeros((b, 1), jnp.float32)
    c202 = jnp.concatenate([zero1, emf, zero1], axis=1)
    e202 = jnp.concatenate([zero1, allzero, h], axis=1)
    code = c202 + 2.0 * e202

    cw = cf_ref.shape[1]
    e_j = lax.broadcasted_iota(jnp.int32, (no, cw), 0)
    e_m = lax.broadcasted_iota(jnp.int32, (no, cw), 1)
    expand = (lax.shift_right_logical(e_m, 4) == e_j).astype(jnp.float32)
    cf_ref[...] = jnp.dot(code, expand, preferred_element_type=jnp.float32)


def _coef_pass(em, cw):
    b, n = em.shape
    return pl.pallas_call(
        _coef_body,
        out_shape=[
            jax.ShapeDtypeStruct((b, n + 2), jnp.int32),
            jax.ShapeDtypeStruct((b, cw), jnp.float32),
        ],
    )(em)


def _sc_stream(x1, cf1, weight, idx, b, n, d, cw):
    info = plsc.get_sparse_core_info()
    nc, ns = info.num_cores, info.num_subcores
    nw = nc * ns
    no = n + 2
    rows_per_w = b // nw
    scale = math.sqrt(d)
    nk = d // 16
    mesh = plsc.VectorSubcoreMesh(core_axis_name="c", subcore_axis_name="s")

    @functools.partial(
        pl.kernel,
        mesh=mesh,
        out_type=jax.ShapeDtypeStruct((b * no * d,), jnp.float32),
        scratch_types=[
            pltpu.VMEM((rows_per_w,), jnp.int32),
            pltpu.VMEM((rows_per_w, d), jnp.float32),
            pltpu.VMEM((8, d), jnp.float32),
            pltpu.VMEM((2 * n * d,), jnp.float32),
            pltpu.VMEM((2 * no * d,), jnp.float32),
            pltpu.VMEM((2 * cw,), jnp.float32),
            pltpu.SemaphoreType.DMA,
            pltpu.SemaphoreType.DMA,
            pltpu.SemaphoreType.DMA,
            pltpu.SemaphoreType.DMA,
            pltpu.SemaphoreType.DMA,
        ],
    )
    def sc_k(x_hbm, cf_hbm, w_hbm, idx_hbm, out_hbm,
             idx_v, lang_v, pe_v, xin_v, xout_v, cfv, gsem, is0, is1, os0, os1):
        wid = lax.axis_index("s") * nc + lax.axis_index("c")
        base = wid * rows_per_w
        in_sems = [is0, is1]
        out_sems = [os0, os1]

        pltpu.sync_copy(idx_hbm.at[pl.ds(base, rows_per_w)], idx_v)
        pltpu.async_copy(w_hbm.at[idx_v], lang_v, gsem).wait()
        pltpu.sync_copy(w_hbm.at[pl.ds(0, 8)], pe_v)
        pad_s = [pe_v[1, pl.ds(k * 16, 16)] * scale for k in range(nk)]
        eos_s = [pe_v[2, pl.ds(k * 16, 16)] * scale for k in range(nk)]

        def start_in(t):
            bg = base + t
            buf = t % 2
            h1 = pltpu.async_copy(
                x_hbm.at[pl.ds(bg * n * d, n * d)],
                xin_v.at[pl.ds(buf * n * d, n * d)], in_sems[buf])
            h2 = pltpu.async_copy(
                cf_hbm.at[pl.ds(bg * cw, cw)],
                cfv.at[pl.ds(buf * cw, cw)], in_sems[buf])
            return (h1, h2)

        h_in = [None, None]
        h_out = [None, None]
        h_in[0] = start_in(0)
        h_in[1] = start_in(1)

        for t in range(rows_per_w):
            buf = t % 2
            h1, h2 = h_in[buf]
            h1.wait()
            h2.wait()
            if h_out[buf] is not None:
                h_out[buf].wait()

            ob = buf * no * d
            ib = buf * n * d
            for k in range(nk):
                xout_v[pl.ds(ob + k * 16, 16)] = (
                    lang_v[t, pl.ds(k * 16, 16)] * scale)

            def jbody(j, _):
                codev = cfv[pl.ds(buf * cw + j * 16, 16)]
                mc = codev == 1.0
                me = codev == 2.0
                for k in range(nk):
                    xv = xin_v[pl.ds(ib + (j - 1) * d + k * 16, 16)]
                    alt = jnp.where(me, eos_s[k], pad_s[k])
                    xout_v[pl.ds(ob + j * d + k * 16, 16)] = jnp.where(
                        mc, xv * scale, alt)
                return 0

            lax.fori_loop(1, no, jbody, 0)

            h_out[buf] = pltpu.async_copy(
                xout_v.at[pl.ds(ob, no * d)],
                out_hbm.at[pl.ds((base + t) * no * d, no * d)], out_sems[buf])
            if t + 2 < rows_per_w:
                h_in[buf] = start_in(t + 2)

        h_out[0].wait()
        h_out[1].wait()

    return sc_k(x1, cf1, weight, idx)


def kernel(x, encoder_padding_mask, src_langtoks, weight):
    b, n, d = x.shape
    cw = ((n + 2) * 16 + 127) // 128 * 128
    idx = src_langtoks.astype(jnp.int32).reshape(b)
    em = encoder_padding_mask.astype(jnp.int32)
    mo, cfef = _coef_pass(em, cw)
    xo = _sc_stream(x.reshape(-1), cfef.reshape(-1), weight, idx, b, n, d, cw)
    return xo.reshape(b, n + 2, d), mo


# speedup vs baseline: 1.6126x; 1.6126x over previous
import functools
import math

import jax
import jax.numpy as jnp
from jax import lax
from jax.experimental import pallas as pl
from jax.experimental.pallas import tpu as pltpu
from jax.experimental.pallas import tpu_sc as plsc

def _coef_body(em_ref, mo_ref, cf_ref):
    b, n = em_ref.shape
    no = n + 2
    em = em_ref[...]
    mo_ref[:, pl.ds(0, 2)] = jnp.ones((b, 2), jnp.int32)
    mo_ref[:, pl.ds(2, n)] = em

    emf = em.astype(jnp.float32)
    r_i = lax.broadcasted_iota(jnp.int32, (n, n), 0)
    c_i = lax.broadcasted_iota(jnp.int32, (n, n), 1)
    tri = (r_i >= c_i).astype(jnp.float32)
    s = jnp.dot(emf, tri, preferred_element_type=jnp.float32)
    h = emf * ((s - emf) == 0.0).astype(jnp.float32)
    allzero = (s[:, 0:1] == 0.0).astype(jnp.float32)

    zero1 = jnp.zeros((b, 1), jnp.float32)
    c202 = jnp.concatenate([zero1, emf, zero1], axis=1)
    e202 = jnp.concatenate([zero1, allzero, h], axis=1)
    code = c202 + 2.0 * e202

    cw = cf_ref.shape[1]
    e_j = lax.broadcasted_iota(jnp.int32, (no, cw), 0)
    e_m = lax.broadcasted_iota(jnp.int32, (no, cw), 1)
    expand = (lax.shift_right_logical(e_m, 4) == e_j).astype(jnp.float32)
    cf_ref[...] = jnp.dot(code, expand, preferred_element_type=jnp.float32)


def _coef_pass(em, cw):
    b, n = em.shape
    return pl.pallas_call(
        _coef_body,
        out_shape=[
            jax.ShapeDtypeStruct((b, n + 2), jnp.int32),
            jax.ShapeDtypeStruct((b, cw), jnp.float32),
        ],
    )(em)


def _sc_stream(x3, cf1, weight, idx, b, n, d, cw):
    info = plsc.get_sparse_core_info()
    nc, ns = info.num_cores, info.num_subcores
    nw = nc * ns
    no = n + 2
    rows_per_w = b // nw
    scale = math.sqrt(d)
    nk = d // 16
    mesh = plsc.VectorSubcoreMesh(core_axis_name="c", subcore_axis_name="s")

    @functools.partial(
        pl.kernel,
        mesh=mesh,
        out_type=jax.ShapeDtypeStruct((b, no, d), jnp.float32),
        scratch_types=[
            pltpu.VMEM((rows_per_w,), jnp.int32),
            pltpu.VMEM((rows_per_w, d), jnp.float32),
            pltpu.VMEM((8, d), jnp.float32),
            pltpu.VMEM((2, n, d), jnp.float32),
            pltpu.VMEM((2, no, d), jnp.float32),
            pltpu.VMEM((2 * cw,), jnp.float32),
            pltpu.SemaphoreType.DMA,
            pltpu.SemaphoreType.DMA,
            pltpu.SemaphoreType.DMA,
            pltpu.SemaphoreType.DMA,
            pltpu.SemaphoreType.DMA,
        ],
    )
    def sc_k(x_hbm, cf_hbm, w_hbm, idx_hbm, out_hbm,
             idx_v, lang_v, pe_v, xin_v, xout_v, cfv, gsem, is0, is1, os0, os1):
        wid = lax.axis_index("s") * nc + lax.axis_index("c")
        base = wid * rows_per_w
        in_sems = [is0, is1]
        out_sems = [os0, os1]

        pltpu.sync_copy(idx_hbm.at[pl.ds(base, rows_per_w)], idx_v)
        pltpu.async_copy(w_hbm.at[idx_v], lang_v, gsem).wait()
        pltpu.sync_copy(w_hbm.at[pl.ds(0, 8)], pe_v)
        pad_s = [pe_v[1, pl.ds(k * 16, 16)] * scale for k in range(nk)]
        eos_s = [pe_v[2, pl.ds(k * 16, 16)] * scale for k in range(nk)]

        def start_in(t):
            bg = base + t
            buf = t % 2
            h1 = pltpu.async_copy(
                x_hbm.at[bg], xin_v.at[buf], in_sems[buf])
            h2 = pltpu.async_copy(
                cf_hbm.at[pl.ds(bg * cw, cw)],
                cfv.at[pl.ds(buf * cw, cw)], in_sems[buf])
            return (h1, h2)

        h_in = [None, None]
        h_out = [None, None]
        h_in[0] = start_in(0)
        h_in[1] = start_in(1)

        for t in range(rows_per_w):
            buf = t % 2
            h1, h2 = h_in[buf]
            h1.wait()
            h2.wait()
            if h_out[buf] is not None:
                h_out[buf].wait()

            for k in range(nk):
                xout_v[buf, 0, pl.ds(k * 16, 16)] = (
                    lang_v[t, pl.ds(k * 16, 16)] * scale)

            def jbody(j, _):
                codev = cfv[pl.ds(buf * cw + j * 16, 16)]
                mc = codev == 1.0
                me = codev == 2.0
                for k in range(nk):
                    xv = xin_v[buf, j - 1, pl.ds(k * 16, 16)]
                    alt = jnp.where(me, eos_s[k], pad_s[k])
                    xout_v[buf, j, pl.ds(k * 16, 16)] = jnp.where(
                        mc, xv * scale, alt)
                return 0

            lax.fori_loop(1, no, jbody, 0)

            h_out[buf] = pltpu.async_copy(
                xout_v.at[buf], out_hbm.at[base + t], out_sems[buf])
            if t + 2 < rows_per_w:
                h_in[buf] = start_in(t + 2)

        h_out[0].wait()
        h_out[1].wait()

    return sc_k(x3, cf1, weight, idx)


def kernel(x, encoder_padding_mask, src_langtoks, weight):
    b, n, d = x.shape
    cw = ((n + 2) * 16 + 127) // 128 * 128
    idx = src_langtoks.astype(jnp.int32).reshape(b)
    em = encoder_padding_mask.astype(jnp.int32)
    mo, cfef = _coef_pass(em, cw)
    xo = _sc_stream(x, cfef.reshape(-1), weight, idx, b, n, d, cw)
    return xo, mo
